# Initial kernel scaffold; baseline (speedup 1.0000x reference)
import jax
import jax.numpy as jnp
from jax.experimental import pallas as pl


def kernel(x, pos, edge_index, batch, Wp, bp, W0, b0, W1, b1, W2, b2, W3, b3):
    raise NotImplementedError("write your pallas kernel here")


# speedup vs baseline: 4.4268x; 4.4268x over previous
import jax
import jax.numpy as jnp
from jax import lax
from jax.experimental import pallas as pl
from jax.experimental.pallas import tpu as pltpu
from jax.experimental.pallas import tpu_sc as plsc

N = 10000
E = 320000
D = 64
DP = 128
NODE_DIM = 128
POS_DIM = 3

NUM_CORES = 2
NUM_SUBCORES = 16
NTILES = NUM_CORES * NUM_SUBCORES
CH = 128
NCH = 79
EPT = CH * NCH
EPAD = NTILES * EPT
NPAD = 10112
ZROWS = NPAD // NUM_SUBCORES
LAST_ROWS = N - 15 * ZROWS

_MESH = plsc.VectorSubcoreMesh(
    core_axis_name="c", subcore_axis_name="s",
    num_cores=NUM_CORES, num_subcores=NUM_SUBCORES)


def _sc_body(h_hbm, src_hbm, dst_hbm, z_hbm, agg_out,
             src_v, dst_v, rows_v, agg_sh, sem):
  c = lax.axis_index("c")
  s = lax.axis_index("s")
  wid = s * NUM_CORES + c

  pltpu.sync_copy(src_hbm.at[wid], src_v)
  pltpu.sync_copy(dst_hbm.at[wid], dst_v)

  zsl = pl.ds(s * ZROWS, ZROWS)
  pltpu.sync_copy(z_hbm.at[zsl], agg_sh.at[zsl])
  plsc.subcore_barrier()

  def chunk(i, carry):
    pltpu.async_copy(h_hbm.at[src_v.at[i]], rows_v, sem).wait()
    pltpu.sync_copy(rows_v, agg_sh.at[dst_v.at[i]], add=True)
    return carry
  lax.fori_loop(0, NCH, chunk, 0)

  plsc.subcore_barrier()

  @pl.when(s < NUM_SUBCORES - 1)
  def _():
    osl = pl.ds(s * ZROWS, ZROWS)
    pltpu.sync_copy(agg_sh.at[osl], agg_out.at[c, osl])

  @pl.when(s == NUM_SUBCORES - 1)
  def _():
    osl = pl.ds(s * ZROWS, LAST_ROWS)
    pltpu.sync_copy(agg_sh.at[osl], agg_out.at[c, osl])


_sc_agg = pl.kernel(
    _sc_body,
    out_type=jax.ShapeDtypeStruct((NUM_CORES, N, DP), jnp.float32),
    mesh=_MESH,
    scratch_types=[
        pltpu.VMEM((NCH, CH), jnp.int32),
        pltpu.VMEM((NCH, CH), jnp.int32),
        pltpu.VMEM((CH, DP), jnp.float32),
        pltpu.VMEM_SHARED((NPAD, DP), jnp.float32),
        pltpu.SemaphoreType.DMA,
    ],
)


def _pad_cols(vals):
  r = vals.shape[0]
  return jnp.concatenate(
      [vals, jnp.ones((r, 1), jnp.float32), jnp.zeros((r, DP - D - 1), jnp.float32)],
      axis=1)


def _proj_body(x_ref, pos_ref, wx_ref, wp_ref, b_ref, o_ref):
  acc = lax.dot_general(
      x_ref[...], wx_ref[...], (((1,), (0,)), ((), ())),
      precision=lax.Precision.HIGHEST, preferred_element_type=jnp.float32)
  acc += lax.dot_general(
      pos_ref[...], wp_ref[...], (((1,), (0,)), ((), ())),
      precision=lax.Precision.HIGHEST, preferred_element_type=jnp.float32)
  o_ref[...] = _pad_cols(acc + b_ref[...])


def _layer_body(h_ref, agg_ref, w_ref, b_ref, o_ref):
  agg = agg_ref[0] + agg_ref[1]
  deg = jnp.maximum(agg[:, D:D + 1], 1.0)
  m = h_ref[:, :D] + agg[:, :D] / deg
  out = lax.dot_general(
      m, w_ref[...], (((1,), (0,)), ((), ())),
      precision=lax.Precision.HIGHEST, preferred_element_type=jnp.float32)
  out = out + b_ref[...]
  o_ref[...] = _pad_cols(jnp.where(out >= 0.0, out, 0.01 * out))


_RB = 2000


def _proj(x, pos, wx, wp, b):
  return pl.pallas_call(
      _proj_body,
      grid=(N // _RB,),
      in_specs=[
          pl.BlockSpec((_RB, NODE_DIM), lambda i: (i, 0)),
          pl.BlockSpec((_RB, POS_DIM), lambda i: (i, 0)),
          pl.BlockSpec((NODE_DIM, D), lambda i: (0, 0)),
          pl.BlockSpec((POS_DIM, D), lambda i: (0, 0)),
          pl.BlockSpec((1, D), lambda i: (0, 0)),
      ],
      out_specs=pl.BlockSpec((_RB, DP), lambda i: (i, 0)),
      out_shape=jax.ShapeDtypeStruct((N, DP), jnp.float32),
  )(x, pos, wx, wp, b)


def _layer(h, agg, w, b):
  return pl.pallas_call(
      _layer_body,
      grid=(N // _RB,),
      in_specs=[
          pl.BlockSpec((_RB, DP), lambda i: (i, 0)),
          pl.BlockSpec((NUM_CORES, _RB, DP), lambda i: (0, i, 0)),
          pl.BlockSpec((D, D), lambda i: (0, 0)),
          pl.BlockSpec((1, D), lambda i: (0, 0)),
      ],
      out_specs=pl.BlockSpec((_RB, DP), lambda i: (i, 0)),
      out_shape=jax.ShapeDtypeStruct((N, DP), jnp.float32),
  )(h, agg, w, b)


def kernel(x, pos, edge_index, batch, Wp, bp, W0, b0, W1, b1, W2, b2, W3, b3):
  del batch, W3, b3
  src = edge_index[0]
  dst = edge_index[1]
  pad = EPAD - E
  src3 = jnp.concatenate(
      [src, jnp.zeros((pad,), jnp.int32)]).reshape(NTILES, NCH, CH)
  dst3 = jnp.concatenate(
      [dst, jnp.full((pad,), N, jnp.int32)]).reshape(NTILES, NCH, CH)
  z = jnp.zeros((NPAD, DP), jnp.float32)

  wx = Wp[POS_DIM:]
  wp = Wp[:POS_DIM]

  h0 = _proj(x, pos, wx, wp, bp.reshape(1, D))
  agg1 = _sc_agg(h0, src3, dst3, z)
  h1 = _layer(h0, agg1, W0, b0.reshape(1, D))
  agg2 = _sc_agg(h1, src3, dst3, z)
  h2 = _layer(h1, agg2, W1, b1.reshape(1, D))
  agg3 = _sc_agg(h2, src3, dst3, z)
  h3 = _layer(h2, agg3, W2, b2.reshape(1, D))
  return jnp.concatenate([h1[:, :D], h2[:, :D], h3[:, :D]], axis=-1)
